# Initial kernel scaffold; baseline (speedup 1.0000x reference)
import jax
import jax.numpy as jnp
from jax.experimental import pallas as pl


def kernel(h_masks, label):
    raise NotImplementedError("write your pallas kernel here")


# speedup vs baseline: 76.9473x; 76.9473x over previous
import jax
import jax.numpy as jnp
from jax import lax
from jax.experimental import pallas as pl
from jax.experimental.pallas import tpu as pltpu

_IM_DIMX = 1333
_IM_DIMY = 800


def _partial_sums_body(x_ref, out_ref):
    out_ref[pl.program_id(0), pl.program_id(1)] = jnp.sum(x_ref[...])


def _make_finalize_body(b, h, w, L, N):
    sx = float(w) / _IM_DIMX
    sy = float(h) / _IM_DIMY
    tn = float(b * h * w)

    def _finalize_body(label_ref, part_ref, out_ref):
        lbl = label_ref[...].astype(jnp.float32)
        x1 = jnp.clip(jnp.round(lbl[:, 0] * sx), 0.0, float(w - 1))
        y1 = jnp.clip(jnp.round(lbl[:, 1] * sy), 0.0, float(h - 1))
        x2 = jnp.clip(jnp.round(lbl[:, 2] * sx), 0.0, float(w))
        y2 = jnp.clip(jnp.round(lbl[:, 3] * sy), 0.0, float(h))
        valid = jnp.logical_not(
            (x2 <= x1) | (y2 <= y1) | (x1 + x2 >= float(w)) | (y1 + y2 >= float(h))
        )
        vf = valid.astype(jnp.float32)

        x1i = x1.astype(jnp.int32)
        y1i = y1.astype(jnp.int32)
        x2i = x2.astype(jnp.int32)
        y2i = y2.astype(jnp.int32)
        rows = lax.broadcasted_iota(jnp.int32, (h, N), 0)
        rt = ((rows >= y1i[None, :]) & (rows < y2i[None, :])).astype(jnp.float32)
        rt = rt * vf[None, :]
        cols = lax.broadcasted_iota(jnp.int32, (N, w), 1)
        cm = ((cols >= x1i[:, None]) & (cols < x2i[:, None])).astype(jnp.float32)
        count = lax.dot_general(
            rt, cm, (((1,), (0,)), ((), ())), preferred_element_type=jnp.float32
        )
        covered = jnp.sum((count > 0.5).astype(jnp.float32))
        pi = covered / tn

        acc = 0.0
        for i in range(L):
            s_i = part_ref[i, 0]
            for j in range(1, b):
                s_i = s_i + part_ref[i, j]
            acc = acc + (s_i / tn - pi) ** 2
        out_ref[0, 0] = acc / float(L)

    return _finalize_body


def kernel(h_masks, label):
    L, B, C, H, W = h_masks.shape
    K, Nb, _ = label.shape
    N = K * Nb

    partials = pl.pallas_call(
        _partial_sums_body,
        grid=(L, B),
        in_specs=[
            pl.BlockSpec((1, 1, C, H, W), lambda i, j: (i, j, 0, 0, 0)),
        ],
        out_specs=pl.BlockSpec(memory_space=pltpu.SMEM),
        out_shape=jax.ShapeDtypeStruct((L, B), jnp.float32),
    )(h_masks)

    boxes = jnp.reshape(label, (N, 4))
    out = pl.pallas_call(
        _make_finalize_body(B, H, W, L, N),
        in_specs=[
            pl.BlockSpec(memory_space=pltpu.VMEM),
            pl.BlockSpec(memory_space=pltpu.SMEM),
        ],
        out_specs=pl.BlockSpec(memory_space=pltpu.SMEM),
        out_shape=jax.ShapeDtypeStruct((1, 1), jnp.float32),
    )(boxes, partials)
    return out[0, 0]


# speedup vs baseline: 78.2195x; 1.0165x over previous
import jax
import jax.numpy as jnp
from jax import lax
from jax.experimental import pallas as pl
from jax.experimental.pallas import tpu as pltpu

_IM_DIMX = 1333
_IM_DIMY = 800

_BATCH_BLOCK = 1


def _make_body(b, c, h, w, L, N, n_j, batch_blk):
    sx = float(w) / _IM_DIMX
    sy = float(h) / _IM_DIMY
    tn = float(b * h * w)

    def _body(label_ref, x_ref, out_ref, pi_ref, acc_ref):
        i = pl.program_id(0)
        j = pl.program_id(1)

        @pl.when((i == 0) & (j == 0))
        def _rasterize():
            lbl = label_ref[...].astype(jnp.float32)
            x1 = jnp.clip(jnp.round(lbl[:, 0] * sx), 0.0, float(w - 1))
            y1 = jnp.clip(jnp.round(lbl[:, 1] * sy), 0.0, float(h - 1))
            x2 = jnp.clip(jnp.round(lbl[:, 2] * sx), 0.0, float(w))
            y2 = jnp.clip(jnp.round(lbl[:, 3] * sy), 0.0, float(h))
            valid = jnp.logical_not(
                (x2 <= x1) | (y2 <= y1) | (x1 + x2 >= float(w)) | (y1 + y2 >= float(h))
            )
            vf = valid.astype(jnp.float32)
            x1i = x1.astype(jnp.int32)
            y1i = y1.astype(jnp.int32)
            x2i = x2.astype(jnp.int32)
            y2i = y2.astype(jnp.int32)
            rows = lax.broadcasted_iota(jnp.int32, (h, N), 0)
            rt = ((rows >= y1i[None, :]) & (rows < y2i[None, :])).astype(jnp.float32)
            rt = rt * vf[None, :]
            cols = lax.broadcasted_iota(jnp.int32, (N, w), 1)
            cm = ((cols >= x1i[:, None]) & (cols < x2i[:, None])).astype(jnp.float32)
            count = lax.dot_general(
                rt, cm, (((1,), (0,)), ((), ())), preferred_element_type=jnp.float32
            )
            covered = jnp.sum((count > 0.5).astype(jnp.float32))
            pi_ref[0] = covered / tn

        s = jnp.sum(x_ref[...])

        @pl.when(j == 0)
        def _init():
            acc_ref[i] = s

        @pl.when(j != 0)
        def _accum():
            acc_ref[i] = acc_ref[i] + s

        @pl.when((i == L - 1) & (j == n_j - 1))
        def _combine():
            pi = pi_ref[0]
            tot = 0.0
            for k in range(L):
                tot = tot + (acc_ref[k] / tn - pi) ** 2
            out_ref[0, 0] = tot / float(L)

    return _body


def kernel(h_masks, label):
    L, B, C, H, W = h_masks.shape
    K, Nb, _ = label.shape
    N = K * Nb
    bb = _BATCH_BLOCK
    n_j = B // bb

    boxes = jnp.reshape(label, (N, 4))
    out = pl.pallas_call(
        _make_body(B, C, H, W, L, N, n_j, bb),
        grid=(L, n_j),
        in_specs=[
            pl.BlockSpec(memory_space=pltpu.VMEM),
            pl.BlockSpec((1, bb, C, H, W), lambda i, j: (i, j, 0, 0, 0)),
        ],
        out_specs=pl.BlockSpec(memory_space=pltpu.SMEM),
        out_shape=jax.ShapeDtypeStruct((1, 1), jnp.float32),
        scratch_shapes=[
            pltpu.SMEM((1,), jnp.float32),
            pltpu.SMEM((L,), jnp.float32),
        ],
    )(boxes, h_masks)
    return out[0, 0]


# speedup vs baseline: 89.6252x; 1.1458x over previous
import jax
import jax.numpy as jnp
from jax import lax
from jax.experimental import pallas as pl
from jax.experimental.pallas import tpu as pltpu

_IM_DIMX = 1333
_IM_DIMY = 800

_BATCH_BLOCK = 2


def _make_body(b, c, h, w, L, N, n_j, batch_blk):
    sx = float(w) / _IM_DIMX
    sy = float(h) / _IM_DIMY
    tn = float(b * h * w)

    def _body(label_ref, x_ref, out_ref, pi_ref, acc_ref):
        i = pl.program_id(0)
        j = pl.program_id(1)

        @pl.when((i == 0) & (j == 0))
        def _rasterize():
            lbl = label_ref[...].astype(jnp.float32)
            x1 = jnp.clip(jnp.round(lbl[:, 0] * sx), 0.0, float(w - 1))
            y1 = jnp.clip(jnp.round(lbl[:, 1] * sy), 0.0, float(h - 1))
            x2 = jnp.clip(jnp.round(lbl[:, 2] * sx), 0.0, float(w))
            y2 = jnp.clip(jnp.round(lbl[:, 3] * sy), 0.0, float(h))
            valid = jnp.logical_not(
                (x2 <= x1) | (y2 <= y1) | (x1 + x2 >= float(w)) | (y1 + y2 >= float(h))
            )
            vf = valid.astype(jnp.float32)
            x1i = x1.astype(jnp.int32)
            y1i = y1.astype(jnp.int32)
            x2i = x2.astype(jnp.int32)
            y2i = y2.astype(jnp.int32)
            rows = lax.broadcasted_iota(jnp.int32, (h, N), 0)
            rt = ((rows >= y1i[None, :]) & (rows < y2i[None, :])).astype(jnp.float32)
            rt = rt * vf[None, :]
            cols = lax.broadcasted_iota(jnp.int32, (N, w), 1)
            cm = ((cols >= x1i[:, None]) & (cols < x2i[:, None])).astype(jnp.float32)
            count = lax.dot_general(
                rt, cm, (((1,), (0,)), ((), ())), preferred_element_type=jnp.float32
            )
            covered = jnp.sum((count > 0.5).astype(jnp.float32))
            pi_ref[0] = covered / tn

        s = jnp.sum(x_ref[...])

        @pl.when(j == 0)
        def _init():
            acc_ref[i] = s

        @pl.when(j != 0)
        def _accum():
            acc_ref[i] = acc_ref[i] + s

        @pl.when((i == L - 1) & (j == n_j - 1))
        def _combine():
            pi = pi_ref[0]
            tot = 0.0
            for k in range(L):
                tot = tot + (acc_ref[k] / tn - pi) ** 2
            out_ref[0, 0] = tot / float(L)

    return _body


def kernel(h_masks, label):
    L, B, C, H, W = h_masks.shape
    K, Nb, _ = label.shape
    N = K * Nb
    bb = _BATCH_BLOCK
    n_j = B // bb

    boxes = jnp.reshape(label, (N, 4))
    out = pl.pallas_call(
        _make_body(B, C, H, W, L, N, n_j, bb),
        grid=(L, n_j),
        in_specs=[
            pl.BlockSpec(memory_space=pltpu.VMEM),
            pl.BlockSpec((1, bb, C, H, W), lambda i, j: (i, j, 0, 0, 0)),
        ],
        out_specs=pl.BlockSpec(memory_space=pltpu.SMEM),
        out_shape=jax.ShapeDtypeStruct((1, 1), jnp.float32),
        scratch_shapes=[
            pltpu.SMEM((1,), jnp.float32),
            pltpu.SMEM((L,), jnp.float32),
        ],
    )(boxes, h_masks)
    return out[0, 0]


# speedup vs baseline: 93.0305x; 1.0380x over previous
import jax
import jax.numpy as jnp
from jax import lax
from jax.experimental import pallas as pl
from jax.experimental.pallas import tpu as pltpu

_IM_DIMX = 1333
_IM_DIMY = 800

_BATCH_BLOCK = 4


def _make_body(b, c, h, w, L, N, n_j, batch_blk):
    sx = float(w) / _IM_DIMX
    sy = float(h) / _IM_DIMY
    tn = float(b * h * w)

    def _body(label_ref, x_ref, out_ref, pi_ref, acc_ref):
        i = pl.program_id(0)
        j = pl.program_id(1)

        @pl.when((i == 0) & (j == 0))
        def _rasterize():
            lbl = label_ref[...].astype(jnp.float32)
            x1 = jnp.clip(jnp.round(lbl[:, 0] * sx), 0.0, float(w - 1))
            y1 = jnp.clip(jnp.round(lbl[:, 1] * sy), 0.0, float(h - 1))
            x2 = jnp.clip(jnp.round(lbl[:, 2] * sx), 0.0, float(w))
            y2 = jnp.clip(jnp.round(lbl[:, 3] * sy), 0.0, float(h))
            valid = jnp.logical_not(
                (x2 <= x1) | (y2 <= y1) | (x1 + x2 >= float(w)) | (y1 + y2 >= float(h))
            )
            vf = valid.astype(jnp.float32)
            x1i = x1.astype(jnp.int32)
            y1i = y1.astype(jnp.int32)
            x2i = x2.astype(jnp.int32)
            y2i = y2.astype(jnp.int32)
            rows = lax.broadcasted_iota(jnp.int32, (h, N), 0)
            rt = ((rows >= y1i[None, :]) & (rows < y2i[None, :])).astype(jnp.float32)
            rt = rt * vf[None, :]
            cols = lax.broadcasted_iota(jnp.int32, (N, w), 1)
            cm = ((cols >= x1i[:, None]) & (cols < x2i[:, None])).astype(jnp.float32)
            count = lax.dot_general(
                rt, cm, (((1,), (0,)), ((), ())), preferred_element_type=jnp.float32
            )
            covered = jnp.sum((count > 0.5).astype(jnp.float32))
            pi_ref[0] = covered / tn

        s = jnp.sum(x_ref[...])

        @pl.when(j == 0)
        def _init():
            acc_ref[i] = s

        @pl.when(j != 0)
        def _accum():
            acc_ref[i] = acc_ref[i] + s

        @pl.when((i == L - 1) & (j == n_j - 1))
        def _combine():
            pi = pi_ref[0]
            tot = 0.0
            for k in range(L):
                tot = tot + (acc_ref[k] / tn - pi) ** 2
            out_ref[0, 0] = tot / float(L)

    return _body


def kernel(h_masks, label):
    L, B, C, H, W = h_masks.shape
    K, Nb, _ = label.shape
    N = K * Nb
    bb = _BATCH_BLOCK
    n_j = B // bb

    boxes = jnp.reshape(label, (N, 4))
    out = pl.pallas_call(
        _make_body(B, C, H, W, L, N, n_j, bb),
        grid=(L, n_j),
        in_specs=[
            pl.BlockSpec(memory_space=pltpu.VMEM),
            pl.BlockSpec((1, bb, C, H, W), lambda i, j: (i, j, 0, 0, 0)),
        ],
        out_specs=pl.BlockSpec(memory_space=pltpu.SMEM),
        out_shape=jax.ShapeDtypeStruct((1, 1), jnp.float32),
        scratch_shapes=[
            pltpu.SMEM((1,), jnp.float32),
            pltpu.SMEM((L,), jnp.float32),
        ],
    )(boxes, h_masks)
    return out[0, 0]


# speedup vs baseline: 97.4723x; 1.0477x over previous
import jax
import jax.numpy as jnp
from jax import lax
from jax.experimental import pallas as pl
from jax.experimental.pallas import tpu as pltpu

_IM_DIMX = 1333
_IM_DIMY = 800

_BATCH_BLOCK = 2


def _make_body(b, c, h, w, L, N, n_j, batch_blk):
    sx = float(w) / _IM_DIMX
    sy = float(h) / _IM_DIMY
    tn = float(b * h * w)

    def _body(label_ref, x_ref, y_ref, out_ref, pi_ref, acc_ref):
        i = pl.program_id(0)
        j = pl.program_id(1)

        @pl.when((i == 0) & (j == 0))
        def _rasterize():
            lbl = label_ref[...].astype(jnp.float32)
            x1 = jnp.clip(jnp.round(lbl[:, 0] * sx), 0.0, float(w - 1))
            y1 = jnp.clip(jnp.round(lbl[:, 1] * sy), 0.0, float(h - 1))
            x2 = jnp.clip(jnp.round(lbl[:, 2] * sx), 0.0, float(w))
            y2 = jnp.clip(jnp.round(lbl[:, 3] * sy), 0.0, float(h))
            valid = jnp.logical_not(
                (x2 <= x1) | (y2 <= y1) | (x1 + x2 >= float(w)) | (y1 + y2 >= float(h))
            )
            vf = valid.astype(jnp.float32)
            x1i = x1.astype(jnp.int32)
            y1i = y1.astype(jnp.int32)
            x2i = x2.astype(jnp.int32)
            y2i = y2.astype(jnp.int32)
            rows = lax.broadcasted_iota(jnp.int32, (h, N), 0)
            rt = ((rows >= y1i[None, :]) & (rows < y2i[None, :])).astype(jnp.float32)
            rt = rt * vf[None, :]
            cols = lax.broadcasted_iota(jnp.int32, (N, w), 1)
            cm = ((cols >= x1i[:, None]) & (cols < x2i[:, None])).astype(jnp.float32)
            count = lax.dot_general(
                rt, cm, (((1,), (0,)), ((), ())), preferred_element_type=jnp.float32
            )
            covered = jnp.sum((count > 0.5).astype(jnp.float32))
            pi_ref[0] = covered / tn

        s = jnp.sum(x_ref[...]) + jnp.sum(y_ref[...])

        @pl.when(j == 0)
        def _init():
            acc_ref[i] = s

        @pl.when(j != 0)
        def _accum():
            acc_ref[i] = acc_ref[i] + s

        @pl.when((i == L - 1) & (j == n_j - 1))
        def _combine():
            pi = pi_ref[0]
            tot = 0.0
            for k in range(L):
                tot = tot + (acc_ref[k] / tn - pi) ** 2
            out_ref[0, 0] = tot / float(L)

    return _body


def kernel(h_masks, label):
    L, B, C, H, W = h_masks.shape
    K, Nb, _ = label.shape
    N = K * Nb
    bb = _BATCH_BLOCK
    n_j = B // (2 * bb)

    boxes = jnp.reshape(label, (N, 4))
    out = pl.pallas_call(
        _make_body(B, C, H, W, L, N, n_j, bb),
        grid=(L, n_j),
        in_specs=[
            pl.BlockSpec(memory_space=pltpu.VMEM),
            pl.BlockSpec((1, bb, C, H, W), lambda i, j: (i, 2 * j, 0, 0, 0)),
            pl.BlockSpec((1, bb, C, H, W), lambda i, j: (i, 2 * j + 1, 0, 0, 0)),
        ],
        out_specs=pl.BlockSpec(memory_space=pltpu.SMEM),
        out_shape=jax.ShapeDtypeStruct((1, 1), jnp.float32),
        scratch_shapes=[
            pltpu.SMEM((1,), jnp.float32),
            pltpu.SMEM((L,), jnp.float32),
        ],
    )(boxes, h_masks, h_masks)
    return out[0, 0]


# speedup vs baseline: 102.2272x; 1.0488x over previous
import jax
import jax.numpy as jnp
from jax import lax
from jax.experimental import pallas as pl
from jax.experimental.pallas import tpu as pltpu

_IM_DIMX = 1333
_IM_DIMY = 800

_BATCH_BLOCK = 1


def _make_body(b, c, h, w, L, N, n_j, batch_blk):
    sx = float(w) / _IM_DIMX
    sy = float(h) / _IM_DIMY
    tn = float(b * h * w)

    def _body(label_ref, x_ref, y_ref, z_ref, w_ref2, out_ref, pi_ref, acc_ref):
        i = pl.program_id(0)
        j = pl.program_id(1)

        @pl.when((i == 0) & (j == 0))
        def _rasterize():
            lbl = label_ref[...].astype(jnp.float32)
            x1 = jnp.clip(jnp.round(lbl[:, 0] * sx), 0.0, float(w - 1))
            y1 = jnp.clip(jnp.round(lbl[:, 1] * sy), 0.0, float(h - 1))
            x2 = jnp.clip(jnp.round(lbl[:, 2] * sx), 0.0, float(w))
            y2 = jnp.clip(jnp.round(lbl[:, 3] * sy), 0.0, float(h))
            valid = jnp.logical_not(
                (x2 <= x1) | (y2 <= y1) | (x1 + x2 >= float(w)) | (y1 + y2 >= float(h))
            )
            vf = valid.astype(jnp.float32)
            x1i = x1.astype(jnp.int32)
            y1i = y1.astype(jnp.int32)
            x2i = x2.astype(jnp.int32)
            y2i = y2.astype(jnp.int32)
            rows = lax.broadcasted_iota(jnp.int32, (h, N), 0)
            rt = ((rows >= y1i[None, :]) & (rows < y2i[None, :])).astype(jnp.float32)
            rt = rt * vf[None, :]
            cols = lax.broadcasted_iota(jnp.int32, (N, w), 1)
            cm = ((cols >= x1i[:, None]) & (cols < x2i[:, None])).astype(jnp.float32)
            count = lax.dot_general(
                rt, cm, (((1,), (0,)), ((), ())), preferred_element_type=jnp.float32
            )
            covered = jnp.sum((count > 0.5).astype(jnp.float32))
            pi_ref[0] = covered / tn

        s = (jnp.sum(x_ref[...]) + jnp.sum(y_ref[...])
             + jnp.sum(z_ref[...]) + jnp.sum(w_ref2[...]))

        @pl.when(j == 0)
        def _init():
            acc_ref[i] = s

        @pl.when(j != 0)
        def _accum():
            acc_ref[i] = acc_ref[i] + s

        @pl.when((i == L - 1) & (j == n_j - 1))
        def _combine():
            pi = pi_ref[0]
            tot = 0.0
            for k in range(L):
                tot = tot + (acc_ref[k] / tn - pi) ** 2
            out_ref[0, 0] = tot / float(L)

    return _body


def kernel(h_masks, label):
    L, B, C, H, W = h_masks.shape
    K, Nb, _ = label.shape
    N = K * Nb
    bb = _BATCH_BLOCK
    n_j = B // (4 * bb)

    boxes = jnp.reshape(label, (N, 4))
    out = pl.pallas_call(
        _make_body(B, C, H, W, L, N, n_j, bb),
        grid=(L, n_j),
        in_specs=[
            pl.BlockSpec(memory_space=pltpu.VMEM),
            pl.BlockSpec((1, bb, C, H, W), lambda i, j: (i, 4 * j, 0, 0, 0)),
            pl.BlockSpec((1, bb, C, H, W), lambda i, j: (i, 4 * j + 1, 0, 0, 0)),
            pl.BlockSpec((1, bb, C, H, W), lambda i, j: (i, 4 * j + 2, 0, 0, 0)),
            pl.BlockSpec((1, bb, C, H, W), lambda i, j: (i, 4 * j + 3, 0, 0, 0)),
        ],
        out_specs=pl.BlockSpec(memory_space=pltpu.SMEM),
        out_shape=jax.ShapeDtypeStruct((1, 1), jnp.float32),
        scratch_shapes=[
            pltpu.SMEM((1,), jnp.float32),
            pltpu.SMEM((L,), jnp.float32),
        ],
    )(boxes, h_masks, h_masks, h_masks, h_masks)
    return out[0, 0]
